# Initial kernel scaffold; baseline (speedup 1.0000x reference)
import jax
import jax.numpy as jnp
from jax.experimental import pallas as pl


def kernel(x, edge_index, train_edge_id, eps1, W1, b1, W2, b2, g1, bb1, eps2, W3, b3, g2, bb2, Wl, bl, Wf, bf):
    raise NotImplementedError("write your pallas kernel here")


# speedup vs baseline: 3.0134x; 3.0134x over previous
import functools

import jax
import jax.numpy as jnp
from jax import lax
from jax.experimental import pallas as pl
from jax.experimental.pallas import tpu as pltpu
from jax.experimental.pallas import tpu_sc as plsc

_NC = 2
_NS = 16
_NW = _NC * _NS

_F32 = jnp.float32
_I32 = jnp.int32
_BN_INV = 0.9999950000374997


def _largest_div_le(n, cap):
    for z in range(min(cap, n), 0, -1):
        if n % z == 0:
            return z
    return 1


@functools.lru_cache(maxsize=None)
def _make_segsum(n, e, d):
    ew = e // _NW
    ch = 128
    full = ew // ch
    tail = ew - full * ch
    rb = 80
    nbl = n // rb
    nloop = -(-nbl // _NS)
    mesh = plsc.VectorSubcoreMesh(core_axis_name="c", subcore_axis_name="s")

    scratch = [
        pltpu.VMEM_SHARED((n, d), _F32),
        pltpu.VMEM((ch,), _I32),
        pltpu.VMEM((ch,), _I32),
        pltpu.VMEM((ch, d), _F32),
        pltpu.SemaphoreType.DMA,
    ]
    if tail:
        scratch += [
            pltpu.VMEM((tail,), _I32),
            pltpu.VMEM((tail,), _I32),
            pltpu.VMEM((tail, d), _F32),
        ]

    @functools.partial(
        pl.kernel,
        out_type=jax.ShapeDtypeStruct((2 * n, d), _F32),
        mesh=mesh,
        scratch_types=scratch,
    )
    def segsum(table, srcr, dstr, out, acc, src_v, dst_v, rows_v, sem, *tl):
        c = lax.axis_index("c")
        s = lax.axis_index("s")
        wid = s * _NC + c

        def zrow(r, carry):
            for k in range(d // 16):
                rows_v[r, pl.ds(k * 16, 16)] = jnp.zeros((16,), _F32)
            return carry

        lax.fori_loop(0, rb, zrow, 0)

        def zblk(j, carry):
            bid = s + j * _NS

            @pl.when(bid < nbl)
            def _():
                pltpu.sync_copy(rows_v.at[pl.ds(0, rb)],
                                acc.at[pl.ds(bid * rb, rb)])

            return carry

        lax.fori_loop(0, nloop, zblk, 0)
        plsc.subcore_barrier()

        base = wid * ew

        def body(i, carry):
            off = base + i * ch
            pltpu.sync_copy(srcr.at[pl.ds(off, ch)], src_v)
            pltpu.sync_copy(dstr.at[pl.ds(off, ch)], dst_v)
            pltpu.async_copy(table.at[src_v], rows_v, sem).wait()
            pltpu.sync_copy(rows_v, acc.at[dst_v], add=True)
            return carry

        lax.fori_loop(0, full, body, 0)
        if tail:
            src_t, dst_t, rows_t = tl
            off = base + full * ch
            pltpu.sync_copy(srcr.at[pl.ds(off, tail)], src_t)
            pltpu.sync_copy(dstr.at[pl.ds(off, tail)], dst_t)
            pltpu.async_copy(table.at[src_t], rows_t, sem).wait()
            pltpu.sync_copy(rows_t, acc.at[dst_t], add=True)
        plsc.subcore_barrier()

        def wblk(j, carry):
            bid = s + j * _NS

            @pl.when(bid < nbl)
            def _():
                pltpu.sync_copy(acc.at[pl.ds(bid * rb, rb)],
                                out.at[pl.ds(c * n + bid * rb, rb)])

            return carry

        lax.fori_loop(0, nloop, wblk, 0)

    return segsum


@functools.lru_cache(maxsize=None)
def _make_pair_prod(n, e, ntr, d):
    tw = ntr // _NW
    ch = 64
    iters = tw // ch
    mesh = plsc.VectorSubcoreMesh(core_axis_name="c", subcore_axis_name="s")

    @functools.partial(
        pl.kernel,
        out_type=jax.ShapeDtypeStruct((ntr, d), _F32),
        mesh=mesh,
        scratch_types=[
            pltpu.VMEM((ch,), _I32),
            pltpu.VMEM((ch,), _I32),
            pltpu.VMEM((ch,), _I32),
            pltpu.VMEM((ch, d), _F32),
            pltpu.VMEM((ch, d), _F32),
            pltpu.SemaphoreType.DMA,
        ],
    )
    def pair_prod(h3, srcr, dstr, teid, out, teid_v, n0_v, n1_v,
                  rows0, rows1, sem):
        c = lax.axis_index("c")
        s = lax.axis_index("s")
        wid = s * _NC + c
        base = wid * tw

        def body(i, carry):
            off = base + i * ch
            pltpu.sync_copy(teid.at[pl.ds(off, ch)], teid_v)
            pltpu.async_copy(srcr.at[teid_v], n0_v, sem).wait()
            pltpu.async_copy(dstr.at[teid_v], n1_v, sem).wait()
            pltpu.async_copy(h3.at[n0_v], rows0, sem).wait()
            pltpu.async_copy(h3.at[n1_v], rows1, sem).wait()

            def prow(r, cr):
                for k in range(d // 16):
                    sl = pl.ds(k * 16, 16)
                    rows0[r, sl] = rows0[r, sl] * rows1[r, sl]
                return cr

            lax.fori_loop(0, ch, prow, 0)
            pltpu.sync_copy(rows0, out.at[pl.ds(off, ch)])
            return carry

        lax.fori_loop(0, iters, body, 0)

    return pair_prod


def _dot(a, b):
    return jnp.dot(a, b, preferred_element_type=_F32,
                   precision=lax.Precision.HIGHEST)


def _mlp1(x, agg, eps1, W1, b1, W2, b2, g1, bb1):
    n, din = x.shape
    h = W2.shape[0]
    R = 1000
    grid = (n // R,)

    def body(eps_ref, x_ref, a_ref, W1_ref, b1_ref, W2_ref, b2_ref, g1_ref,
             bb1_ref, out_ref):
        sm = (1.0 + eps_ref[0, 0]) * x_ref[...] + a_ref[0] + a_ref[1]
        t = jnp.maximum(_dot(sm, W1_ref[...]) + b1_ref[...], 0.0)
        u = jnp.maximum(_dot(t, W2_ref[...]) + b2_ref[...], 0.0)
        u = u * (g1_ref[...] * _BN_INV) + bb1_ref[...]
        for cidx in range(h // din):
            out_ref[cidx] = u[:, cidx * din:(cidx + 1) * din]

    return pl.pallas_call(
        body,
        grid=grid,
        in_specs=[
            pl.BlockSpec(memory_space=pltpu.SMEM),
            pl.BlockSpec((R, din), lambda i: (i, 0)),
            pl.BlockSpec((2, R, din), lambda i: (0, i, 0)),
            pl.BlockSpec((din, h), lambda i: (0, 0)),
            pl.BlockSpec((1, h), lambda i: (0, 0)),
            pl.BlockSpec((h, h), lambda i: (0, 0)),
            pl.BlockSpec((1, h), lambda i: (0, 0)),
            pl.BlockSpec((1, h), lambda i: (0, 0)),
            pl.BlockSpec((1, h), lambda i: (0, 0)),
        ],
        out_specs=pl.BlockSpec((h // din, R, din), lambda i: (0, i, 0)),
        out_shape=jax.ShapeDtypeStruct((h // din, n, din), _F32),
    )(eps1.reshape(1, 1), x, agg.reshape(2, n, din), W1, b1.reshape(1, h),
      W2, b2.reshape(1, h), g1.reshape(1, h), bb1.reshape(1, h))


def _mlp2(hch, aggs, eps2, W3, b3, g2, bb2, Wl, bl):
    nch, n, din = hch.shape
    h = W3.shape[0]
    R = 1000
    grid = (n // R,)

    def body(eps_ref, h_ref, a0, a1, a2, a3, W3_ref, b3_ref, g2_ref, bb2_ref,
             Wl_ref, bl_ref, out_ref):
        arefs = (a0, a1, a2, a3)
        hh = jnp.concatenate([h_ref[j] for j in range(nch)], axis=1)
        ag = jnp.concatenate([arefs[j][0] + arefs[j][1] for j in range(nch)],
                             axis=1)
        sm = (1.0 + eps_ref[0, 0]) * hh + ag
        h2 = jnp.maximum(_dot(sm, W3_ref[...]) + b3_ref[...], 0.0)
        h2 = h2 * (g2_ref[...] * _BN_INV) + bb2_ref[...]
        out_ref[...] = jnp.maximum(_dot(h2, Wl_ref[...]) + bl_ref[...], 0.0)

    aspec = pl.BlockSpec((2, R, din), lambda i: (0, i, 0))
    wspec = pl.BlockSpec((h, h), lambda i: (0, 0))
    vspec = pl.BlockSpec((1, h), lambda i: (0, 0))
    return pl.pallas_call(
        body,
        grid=grid,
        in_specs=[
            pl.BlockSpec(memory_space=pltpu.SMEM),
            pl.BlockSpec((nch, R, din), lambda i: (0, i, 0)),
            aspec, aspec, aspec, aspec,
            wspec, vspec, vspec, vspec, wspec, vspec,
        ],
        out_specs=pl.BlockSpec((R, h), lambda i: (i, 0)),
        out_shape=jax.ShapeDtypeStruct((n, h), _F32),
    )(eps2.reshape(1, 1), hch,
      aggs[0].reshape(2, n, din), aggs[1].reshape(2, n, din),
      aggs[2].reshape(2, n, din), aggs[3].reshape(2, n, din),
      W3, b3.reshape(1, h), g2.reshape(1, h), bb2.reshape(1, h),
      Wl, bl.reshape(1, h))


def _head(prod, Wf, bf):
    ntr, h = prod.shape
    cdim = Wf.shape[1]
    BT = 4096
    grid = (ntr // BT,)

    def body(p_ref, Wf_ref, bf_ref, out_ref):
        out_ref[...] = _dot(p_ref[...], Wf_ref[...]) + bf_ref[...]

    return pl.pallas_call(
        body,
        grid=grid,
        in_specs=[
            pl.BlockSpec((BT, h), lambda i: (i, 0)),
            pl.BlockSpec((h, cdim), lambda i: (0, 0)),
            pl.BlockSpec((1, cdim), lambda i: (0, 0)),
        ],
        out_specs=pl.BlockSpec((BT, cdim), lambda i: (i, 0)),
        out_shape=jax.ShapeDtypeStruct((ntr, cdim), _F32),
    )(prod, Wf, bf.reshape(1, cdim))


def kernel(x, edge_index, train_edge_id, eps1, W1, b1, W2, b2, g1, bb1,
           eps2, W3, b3, g2, bb2, Wl, bl, Wf, bf):
    n, din = x.shape
    e = edge_index.shape[1]
    ntr = train_edge_id.shape[0]
    h = W2.shape[0]

    src = edge_index[0]
    dst = edge_index[1]

    agg1 = _make_segsum(n, e, din)(x, src, dst)
    hch = _mlp1(x, agg1, eps1, W1, b1, W2, b2, g1, bb1)

    seg2 = _make_segsum(n, e, din)
    aggs = tuple(seg2(hch[j], src, dst) for j in range(h // din))
    h3 = _mlp2(hch, aggs, eps2, W3, b3, g2, bb2, Wl, bl)

    prod = _make_pair_prod(n, e, ntr, h)(h3, src, dst, train_edge_id)
    return _head(prod, Wf, bf)
